# baseline (speedup 1.0000x reference)
import dataclasses
import functools

import jax
import jax.numpy as jnp
from jax import lax
from jax.experimental import pallas as pl
from jax.experimental.pallas import tpu as pltpu
from jax.experimental.pallas import tpu_sc as plsc

B, HW, D = 4, 1024, 768
L = 4096
H, Dh = 8, 64
KN = 32
INNER = H * Dh
NW = 32
QW = HW // NW
NV = INNER // 16


def _fold_cols(w):
    return w.reshape(-1, H, Dh).transpose(0, 2, 1).reshape(-1, INNER)


def _word_cols(w):
    return w.reshape(-1, H, 16, 2, 2).transpose(0, 3, 2, 4, 1).reshape(-1, 2, 256)


def _mm(a, b, bias=None, bm=512, pack_kv=False):
    M, K = a.shape
    _, N = b.shape
    in_specs = [pl.BlockSpec((bm, K), lambda i: (i, 0)),
                pl.BlockSpec((K, N), lambda i: (0, 0))]
    args = [a, b]
    has_bias = bias is not None
    if has_bias:
        in_specs.append(pl.BlockSpec((1, N), lambda i: (0, 0)))
        args.append(bias.reshape(1, N))

    def body(*refs):
        a_ref, b_ref = refs[0], refs[1]
        o_ref = refs[-1]
        acc = lax.dot_general(a_ref[...], b_ref[...], (((1,), (0,)), ((), ())),
                              preferred_element_type=jnp.float32,
                              precision=lax.Precision.DEFAULT)
        if has_bias:
            acc = acc + refs[2][...]
        if pack_kv:
            half = N // 2
            lo = lax.bitcast_convert_type(
                acc[:, :half].astype(jnp.bfloat16), jnp.uint16).astype(jnp.uint32)
            hi = lax.bitcast_convert_type(
                acc[:, half:].astype(jnp.bfloat16), jnp.uint16).astype(jnp.uint32)
            o_ref[...] = lax.bitcast_convert_type(lo | (hi << 16), jnp.int32)
        else:
            o_ref[...] = acc

    out_n = N // 2 if pack_kv else N
    out_dtype = jnp.int32 if pack_kv else jnp.float32
    return pl.pallas_call(
        body,
        grid=(M // bm,),
        in_specs=in_specs,
        out_specs=pl.BlockSpec((bm, out_n), lambda i: (i, 0)),
        out_shape=jax.ShapeDtypeStruct((M, out_n), out_dtype),
    )(*args)


def _rot8(v):
    idx = lax.iota(jnp.int32, 16) ^ 8
    dnums = lax.GatherDimensionNumbers(
        offset_dims=(), collapsed_slice_dims=(0,), start_index_map=(0,))
    return lax.gather(v, idx[:, None], dnums, (1,),
                      mode=lax.GatherScatterMode.PROMISE_IN_BOUNDS)


def _unpack_bf16(xi):
    a = plsc.bitcast(xi << 16, jnp.float32)
    b = plsc.bitcast(xi, jnp.float32)
    return a, b


def _sc_attn(q, kv, idx, bias):
    mesh = plsc.VectorSubcoreMesh(core_axis_name="c", subcore_axis_name="s")
    cp = pltpu.CompilerParams()
    if "needs_layout_passes" in pltpu.CompilerParams.__dataclass_fields__:
        cp = dataclasses.replace(cp, needs_layout_passes=False)

    @functools.partial(
        pl.kernel,
        out_type=jax.ShapeDtypeStruct((HW, INNER), jnp.float32),
        mesh=mesh,
        compiler_params=cp,
        scratch_types=[
            pltpu.VMEM((QW, INNER), jnp.float32),
            pltpu.VMEM((QW * KN,), jnp.int32),
            pltpu.VMEM((QW, KN), jnp.float32),
            pltpu.VMEM((2 * KN, INNER), jnp.int32),
            pltpu.VMEM((2 * KN, INNER), jnp.int32),
            pltpu.VMEM((KN, 16), jnp.float32),
            pltpu.VMEM((QW, INNER), jnp.float32),
            pltpu.SemaphoreType.DMA,
            pltpu.SemaphoreType.DMA,
        ],
    )
    def body(q_hbm, kv_hbm, idx_hbm, bias_hbm, o_hbm,
             qv, idxv, biasv, kvg0, kvg1, simv, outv, sem0, sem1):
        wid = lax.axis_index("s") * 2 + lax.axis_index("c")
        base = wid * QW
        pltpu.sync_copy(idx_hbm.at[pl.ds(base * KN, QW * KN)], idxv)
        pltpu.sync_copy(bias_hbm.at[pl.ds(base, QW)], biasv)
        pltpu.sync_copy(q_hbm.at[pl.ds(base, QW)], qv)

        def start_pair(qi, buf, sem):
            pltpu.async_copy(kv_hbm.at[idxv.at[pl.ds(qi * KN, 2 * KN)]], buf, sem)

        def wait_pair(buf, sem):
            pltpu.make_async_copy(kv_hbm.at[idxv.at[pl.ds(0, 2 * KN)]], buf, sem).wait()

        def tree_sum(terms):
            while len(terms) > 1:
                pairs = [terms[i] + terms[i + 1] for i in range(0, len(terms) - 1, 2)]
                if len(terms) % 2:
                    pairs.append(terms[-1])
                terms = pairs
            return terms[0]

        def compute(t, u, kvg):
            qvecs = [qv[t, pl.ds(16 * j, 16)] for j in range(NV)]
            bvecs = [biasv[t, pl.ds(16 * j, 16)] for j in range(KN // 16)]
            den = None
            for kk in range(KN):
                terms = []
                for j in range(NV // 2):
                    ka, kb = _unpack_bf16(kvg[u * KN + kk, pl.ds(16 * j, 16)])
                    terms.append(qvecs[2 * j] * ka + qvecs[2 * j + 1] * kb)
                p = tree_sum(terms)
                e = jnp.exp(p + _rot8(p) + bvecs[kk // 16][kk % 16])
                simv[kk, :] = e
                den = e if den is None else den + e
            inv = 1.0 / den
            accs = None
            for kk in range(KN):
                w = simv[kk, :]
                term = []
                for j in range(NV // 2):
                    va, vb = _unpack_bf16(
                        kvg[u * KN + kk, pl.ds(INNER // 2 + 16 * j, 16)])
                    term += [w * va, w * vb]
                accs = term if accs is None else [a + v for a, v in zip(accs, term)]
            for j in range(NV):
                outv[t, pl.ds(16 * j, 16)] = accs[j] * inv

        start_pair(0, kvg0, sem0)

        @pl.loop(0, QW, step=4)
        def _(qi):
            @pl.when(qi + 2 < QW)
            def _():
                start_pair(qi + 2, kvg1, sem1)

            wait_pair(kvg0, sem0)

            @pl.loop(0, 2)
            def _(u):
                compute(qi + u, u, kvg0)

            @pl.when(qi + 4 < QW)
            def _():
                start_pair(qi + 4, kvg0, sem0)

            wait_pair(kvg1, sem1)

            @pl.loop(0, 2)
            def _(u):
                compute(qi + 2 + u, u, kvg1)

        pltpu.sync_copy(outv, o_hbm.at[pl.ds(base, QW)])

    return body(q, kv, idx, bias)


def kernel(x, context, attn_indices, bias, Wq, Wkv, Wout, bout):
    scale = Dh ** (-0.5)
    Wq_p = _fold_cols(Wq * scale)
    wk = _word_cols(Wkv[:, :INNER])
    wv = _word_cols(Wkv[:, INNER:])
    Wkv_p = jnp.concatenate(
        [wk[:, 0], wv[:, 0], wk[:, 1], wv[:, 1]], axis=1)
    Wout_p = Wout.reshape(H, Dh, D).transpose(1, 0, 2).reshape(INNER, D)
    idx = attn_indices.astype(jnp.int32).reshape(B, HW * KN)
    bias = bias.astype(jnp.float32)

    outs = []
    for b in range(B):
        qp = _mm(x[b], Wq_p)
        kvp = _mm(context[b], Wkv_p, pack_kv=True)
        attn = _sc_attn(qp, kvp, idx[b], bias[b])
        outs.append(_mm(attn, Wout_p, bias=bout))
    return jnp.stack(outs)


# speedup vs baseline: 1.3714x; 1.3714x over previous
import dataclasses
import functools

import jax
import jax.numpy as jnp
from jax import lax
from jax.experimental import pallas as pl
from jax.experimental.pallas import tpu as pltpu
from jax.experimental.pallas import tpu_sc as plsc

B, HW, D = 4, 1024, 768
L = 4096
H, Dh = 8, 64
KN = 32
INNER = H * Dh
NW = 32
QW = HW // NW
NV = INNER // 16


def _fold_cols(w):
    return w.reshape(-1, H, Dh).transpose(0, 2, 1).reshape(-1, INNER)


def _word_cols(w):
    return w.reshape(-1, H, 16, 2, 2).transpose(0, 3, 2, 4, 1).reshape(-1, 2, 256)


def _mm(a, b, bias=None, bm=512, pack_kv=False):
    M, K = a.shape
    _, N = b.shape
    in_specs = [pl.BlockSpec((bm, K), lambda i: (i, 0)),
                pl.BlockSpec((K, N), lambda i: (0, 0))]
    args = [a, b]
    has_bias = bias is not None
    if has_bias:
        in_specs.append(pl.BlockSpec((1, N), lambda i: (0, 0)))
        args.append(bias.reshape(1, N))

    def body(*refs):
        a_ref, b_ref = refs[0], refs[1]
        o_ref = refs[-1]
        acc = lax.dot_general(a_ref[...], b_ref[...], (((1,), (0,)), ((), ())),
                              preferred_element_type=jnp.float32,
                              precision=lax.Precision.DEFAULT)
        if has_bias:
            acc = acc + refs[2][...]
        if pack_kv:
            half = N // 2
            lo = lax.bitcast_convert_type(
                acc[:, :half].astype(jnp.bfloat16), jnp.uint16).astype(jnp.uint32)
            hi = lax.bitcast_convert_type(
                acc[:, half:].astype(jnp.bfloat16), jnp.uint16).astype(jnp.uint32)
            o_ref[...] = lax.bitcast_convert_type(lo | (hi << 16), jnp.int32)
        else:
            o_ref[...] = acc

    out_n = N // 2 if pack_kv else N
    out_dtype = jnp.int32 if pack_kv else jnp.float32
    return pl.pallas_call(
        body,
        grid=(M // bm,),
        in_specs=in_specs,
        out_specs=pl.BlockSpec((bm, out_n), lambda i: (i, 0)),
        out_shape=jax.ShapeDtypeStruct((M, out_n), out_dtype),
    )(*args)


def _rot8(v):
    idx = lax.iota(jnp.int32, 16) ^ 8
    dnums = lax.GatherDimensionNumbers(
        offset_dims=(), collapsed_slice_dims=(0,), start_index_map=(0,))
    return lax.gather(v, idx[:, None], dnums, (1,),
                      mode=lax.GatherScatterMode.PROMISE_IN_BOUNDS)


def _unpack_bf16(xi):
    a = plsc.bitcast(xi << 16, jnp.float32)
    b = plsc.bitcast(xi, jnp.float32)
    return a, b


def _sc_attn(q, kv, idx, bias):
    mesh = plsc.VectorSubcoreMesh(core_axis_name="c", subcore_axis_name="s")
    cp = pltpu.CompilerParams()
    if "needs_layout_passes" in pltpu.CompilerParams.__dataclass_fields__:
        cp = dataclasses.replace(cp, needs_layout_passes=False)

    @functools.partial(
        pl.kernel,
        out_type=jax.ShapeDtypeStruct((HW, INNER), jnp.float32),
        mesh=mesh,
        compiler_params=cp,
        scratch_types=[
            pltpu.VMEM((QW, INNER), jnp.float32),
            pltpu.VMEM((QW, KN), jnp.int32),
            pltpu.VMEM((QW, KN), jnp.float32),
            pltpu.VMEM((KN, INNER), jnp.int32),
            pltpu.VMEM((KN, INNER), jnp.int32),
            pltpu.VMEM((KN, 16), jnp.float32),
            pltpu.VMEM((QW, INNER), jnp.float32),
            pltpu.SemaphoreType.DMA,
            pltpu.SemaphoreType.DMA,
        ],
    )
    def body(q_hbm, kv_hbm, idx_hbm, bias_hbm, o_hbm,
             qv, idxv, biasv, kvg0, kvg1, simv, outv, sem0, sem1):
        wid = lax.axis_index("s") * 2 + lax.axis_index("c")
        base = wid * QW
        pltpu.sync_copy(idx_hbm.at[pl.ds(base, QW)], idxv)
        pltpu.sync_copy(bias_hbm.at[pl.ds(base, QW)], biasv)
        pltpu.sync_copy(q_hbm.at[pl.ds(base, QW)], qv)

        def start_gather(qi, buf, sem):
            pltpu.async_copy(kv_hbm.at[idxv.at[qi]], buf, sem)

        def wait_gather(buf, sem):
            pltpu.make_async_copy(kv_hbm.at[idxv.at[0]], buf, sem).wait()

        def tree_sum(terms):
            while len(terms) > 1:
                pairs = [terms[i] + terms[i + 1] for i in range(0, len(terms) - 1, 2)]
                if len(terms) % 2:
                    pairs.append(terms[-1])
                terms = pairs
            return terms[0]

        def compute(t, kvg):
            qvecs = [qv[t, pl.ds(16 * j, 16)] for j in range(NV)]
            bvecs = [biasv[t, pl.ds(16 * j, 16)] for j in range(KN // 16)]
            den = None
            for kk in range(KN):
                terms = []
                for j in range(NV // 2):
                    ka, kb = _unpack_bf16(kvg[kk, pl.ds(16 * j, 16)])
                    terms.append(qvecs[2 * j] * ka + qvecs[2 * j + 1] * kb)
                p = tree_sum(terms)
                e = jnp.exp(p + _rot8(p) + bvecs[kk // 16][kk % 16])
                simv[kk, :] = e
                den = e if den is None else den + e
            inv = 1.0 / den
            accs = None
            for kk in range(KN):
                w = simv[kk, :]
                term = []
                for j in range(NV // 2):
                    va, vb = _unpack_bf16(kvg[kk, pl.ds(INNER // 2 + 16 * j, 16)])
                    term += [w * va, w * vb]
                accs = term if accs is None else [a + v for a, v in zip(accs, term)]
            for j in range(NV):
                outv[t, pl.ds(16 * j, 16)] = accs[j] * inv

        start_gather(0, kvg0, sem0)

        @pl.loop(0, QW, step=2)
        def _(qi):
            start_gather(qi + 1, kvg1, sem1)
            wait_gather(kvg0, sem0)
            compute(qi, kvg0)

            @pl.when(qi + 2 < QW)
            def _():
                start_gather(qi + 2, kvg0, sem0)

            wait_gather(kvg1, sem1)
            compute(qi + 1, kvg1)

        pltpu.sync_copy(outv, o_hbm.at[pl.ds(base, QW)])

    return body(q, kv, idx, bias)


def kernel(x, context, attn_indices, bias, Wq, Wkv, Wout, bout):
    scale = Dh ** (-0.5)
    Wq_p = _fold_cols(Wq * scale)
    wk = _word_cols(Wkv[:, :INNER])
    wv = _word_cols(Wkv[:, INNER:])
    Wkv_p = jnp.concatenate(
        [wk[:, 0], wv[:, 0], wk[:, 1], wv[:, 1]], axis=1)
    Wout_p = Wout.reshape(H, Dh, D).transpose(1, 0, 2).reshape(INNER, D)
    idx = attn_indices.astype(jnp.int32)
    bias = bias.astype(jnp.float32)

    outs = []
    for b in range(B):
        qp = _mm(x[b], Wq_p)
        kvp = _mm(context[b], Wkv_p, pack_kv=True)
        attn = _sc_attn(qp, kvp, idx[b], bias[b])
        outs.append(_mm(attn, Wout_p, bias=bout))
    return jnp.stack(outs)


# speedup vs baseline: 1.5241x; 1.1114x over previous
import dataclasses
import functools

import jax
import jax.numpy as jnp
from jax import lax
from jax.experimental import pallas as pl
from jax.experimental.pallas import tpu as pltpu
from jax.experimental.pallas import tpu_sc as plsc

B, HW, D = 4, 1024, 768
L = 4096
H, Dh = 8, 64
KN = 32
INNER = H * Dh
NW = 32
QW = HW // NW
NV = INNER // 16


def _fold_cols(w):
    return w.reshape(-1, H, Dh).transpose(0, 2, 1).reshape(-1, INNER)


def _word_cols(w):
    return w.reshape(-1, H, 16, 2, 2).transpose(0, 3, 2, 4, 1).reshape(-1, 2, 256)


def _mm(a, b, bias=None, bm=512, pack_kv=False, batch=None):
    if batch is None:
        M, K = a.shape
        a_spec = pl.BlockSpec((bm, K), lambda i: (i, 0))
    else:
        _, M, K = a.shape
        a_spec = pl.BlockSpec((1, bm, K), lambda i, _b=batch: (_b, i, 0))
    _, N = b.shape
    in_specs = [a_spec, pl.BlockSpec((K, N), lambda i: (0, 0))]
    args = [a, b]
    has_bias = bias is not None
    if has_bias:
        in_specs.append(pl.BlockSpec((1, N), lambda i: (0, 0)))
        args.append(bias.reshape(1, N))

    def body(*refs):
        a_ref, b_ref = refs[0], refs[1]
        o_ref = refs[-1]
        av = a_ref[...] if batch is None else a_ref[0]
        acc = lax.dot_general(av, b_ref[...], (((1,), (0,)), ((), ())),
                              preferred_element_type=jnp.float32,
                              precision=lax.Precision.DEFAULT)
        if has_bias:
            acc = acc + refs[2][...]
        if pack_kv:
            half = N // 2
            lo = lax.bitcast_convert_type(
                acc[:, :half].astype(jnp.bfloat16), jnp.uint16).astype(jnp.uint32)
            hi = lax.bitcast_convert_type(
                acc[:, half:].astype(jnp.bfloat16), jnp.uint16).astype(jnp.uint32)
            o_ref[...] = lax.bitcast_convert_type(lo | (hi << 16), jnp.int32)
        else:
            o_ref[...] = acc

    out_n = N // 2 if pack_kv else N
    out_dtype = jnp.int32 if pack_kv else jnp.float32
    return pl.pallas_call(
        body,
        grid=(M // bm,),
        in_specs=in_specs,
        out_specs=pl.BlockSpec((bm, out_n), lambda i: (i, 0)),
        out_shape=jax.ShapeDtypeStruct((M, out_n), out_dtype),
    )(*args)


def _rot8(v):
    idx = lax.iota(jnp.int32, 16) ^ 8
    dnums = lax.GatherDimensionNumbers(
        offset_dims=(), collapsed_slice_dims=(0,), start_index_map=(0,))
    return lax.gather(v, idx[:, None], dnums, (1,),
                      mode=lax.GatherScatterMode.PROMISE_IN_BOUNDS)


def _unpack_bf16(xi):
    a = plsc.bitcast(xi << 16, jnp.float32)
    b = plsc.bitcast(xi, jnp.float32)
    return a, b


def _sc_attn(q, kv, idx, bias):
    mesh = plsc.VectorSubcoreMesh(core_axis_name="c", subcore_axis_name="s")
    cp = pltpu.CompilerParams()
    if "needs_layout_passes" in pltpu.CompilerParams.__dataclass_fields__:
        cp = dataclasses.replace(cp, needs_layout_passes=False)

    @functools.partial(
        pl.kernel,
        out_type=jax.ShapeDtypeStruct((HW, INNER), jnp.float32),
        mesh=mesh,
        compiler_params=cp,
        scratch_types=[
            pltpu.VMEM((QW, INNER), jnp.float32),
            pltpu.VMEM((QW, KN), jnp.int32),
            pltpu.VMEM((QW, KN), jnp.float32),
            pltpu.VMEM((KN, INNER), jnp.int32),
            pltpu.VMEM((KN, INNER), jnp.int32),
            pltpu.VMEM((KN, 16), jnp.float32),
            pltpu.VMEM((QW, INNER), jnp.float32),
            pltpu.SemaphoreType.DMA,
            pltpu.SemaphoreType.DMA,
            pltpu.SemaphoreType.DMA,
            pltpu.SemaphoreType.DMA,
        ],
    )
    def body(q_hbm, kv_hbm, idx_hbm, bias_hbm, o_hbm,
             qv, idxv, biasv, kvg0, kvg1, simv, outv, sem0, sem1, semq, semb):
        wid = lax.axis_index("s") * 2 + lax.axis_index("c")
        base = wid * QW
        pltpu.sync_copy(idx_hbm.at[pl.ds(base, QW)], idxv)
        q_cp = pltpu.async_copy(q_hbm.at[pl.ds(base, QW)], qv, semq)
        b_cp = pltpu.async_copy(bias_hbm.at[pl.ds(base, QW)], biasv, semb)

        def start_gather(qi, buf, sem):
            pltpu.async_copy(kv_hbm.at[idxv.at[qi]], buf, sem)

        def wait_gather(buf, sem):
            pltpu.make_async_copy(kv_hbm.at[idxv.at[0]], buf, sem).wait()

        def tree_sum(terms):
            while len(terms) > 1:
                pairs = [terms[i] + terms[i + 1] for i in range(0, len(terms) - 1, 2)]
                if len(terms) % 2:
                    pairs.append(terms[-1])
                terms = pairs
            return terms[0]

        def compute(t, kvg):
            qvecs = [qv[t, pl.ds(16 * j, 16)] for j in range(NV)]
            bvecs = [biasv[t, pl.ds(16 * j, 16)] for j in range(KN // 16)]
            den = None
            for kk in range(KN):
                terms = []
                for j in range(NV // 2):
                    ka, kb = _unpack_bf16(kvg[kk, pl.ds(16 * j, 16)])
                    terms.append(qvecs[2 * j] * ka + qvecs[2 * j + 1] * kb)
                p = tree_sum(terms)
                e = jnp.exp(p + _rot8(p) + bvecs[kk // 16][kk % 16])
                simv[kk, :] = e
                den = e if den is None else den + e
            inv = 1.0 / den
            accs = None
            for kk in range(KN):
                w = simv[kk, :]
                term = []
                for j in range(NV // 2):
                    va, vb = _unpack_bf16(kvg[kk, pl.ds(INNER // 2 + 16 * j, 16)])
                    term += [w * va, w * vb]
                accs = term if accs is None else [a + v for a, v in zip(accs, term)]
            for j in range(NV):
                outv[t, pl.ds(16 * j, 16)] = accs[j] * inv

        start_gather(0, kvg0, sem0)
        q_cp.wait()
        b_cp.wait()

        @pl.loop(0, QW, step=2)
        def _(qi):
            start_gather(qi + 1, kvg1, sem1)
            wait_gather(kvg0, sem0)
            compute(qi, kvg0)

            @pl.when(qi + 2 < QW)
            def _():
                start_gather(qi + 2, kvg0, sem0)

            wait_gather(kvg1, sem1)
            compute(qi + 1, kvg1)

        pltpu.sync_copy(outv, o_hbm.at[pl.ds(base, QW)])

    return body(q, kv, idx, bias)


def kernel(x, context, attn_indices, bias, Wq, Wkv, Wout, bout):
    scale = Dh ** (-0.5)
    Wq_p = _fold_cols(Wq * scale)
    wk = _word_cols(Wkv[:, :INNER])
    wv = _word_cols(Wkv[:, INNER:])
    Wkv_p = jnp.concatenate(
        [wk[:, 0], wv[:, 0], wk[:, 1], wv[:, 1]], axis=1)
    Wout_p = Wout.reshape(H, Dh, D).transpose(1, 0, 2).reshape(INNER, D)
    idx = attn_indices.astype(jnp.int32)
    bias = bias.astype(jnp.float32)

    outs = []
    for b in range(B):
        qp = _mm(x, Wq_p, batch=b)
        kvp = _mm(context, Wkv_p, pack_kv=True, batch=b)
        attn = _sc_attn(qp, kvp, idx[b], bias[b])
        outs.append(_mm(attn, Wout_p, bias=bout))
    return jnp.stack(outs)
